# baseline (speedup 1.0000x reference)
import jax
import jax.numpy as jnp
from jax import lax
from jax.experimental import pallas as pl
from jax.experimental.pallas import tpu as pltpu
from jax.experimental.pallas import tpu_sc as plsc

NC = 2
NS = 16
NW = NC * NS
LANES = 16
CH = 128


def _largest_div(n, limit, mult):
    best = mult
    for d in range(mult, limit + 1, mult):
        if n % d == 0:
            best = d
    return best


def _make_sc_agg(N, E, D, with_counts):
    NCHT = E // CH
    Q, R_ = divmod(NCHT, NW)
    PQ = Q // 2
    EXTRA = (Q - 2 * PQ) + (1 if R_ else 0)
    RSTEP = (N // NS) // 8 * 8
    RLEN = N - (NS - 1) * RSTEP
    ZR = _largest_div(RLEN, 16, 8)
    NZ = RLEN // ZR
    PB = _largest_div(RLEN, CH, 8)
    NPUB = RLEN // PB

    mesh = plsc.VectorSubcoreMesh(core_axis_name="c", subcore_axis_name="s",
                                  num_cores=NC, num_subcores=NS)

    out_type = [jax.ShapeDtypeStruct((NC, N, D), jnp.float32)]
    if with_counts:
        out_type.append(jax.ShapeDtypeStruct((NC * N,), jnp.float32))

    scratch = [
        pltpu.VMEM((2, CH), jnp.int32),
        pltpu.VMEM((2, CH), jnp.int32),
        pltpu.VMEM((CH,), jnp.int32),
        pltpu.VMEM((CH,), jnp.int32),
        pltpu.VMEM((CH,), jnp.int32),
        pltpu.VMEM((CH,), jnp.int32),
        pltpu.VMEM((CH, D), jnp.float32),
        pltpu.VMEM((CH, D), jnp.float32),
        pltpu.VMEM((ZR, D), jnp.float32),
        pltpu.VMEM_SHARED((N, D), jnp.float32),
        pltpu.SemaphoreType.DMA,
        pltpu.SemaphoreType.DMA,
        pltpu.SemaphoreType.DMA,
        pltpu.SemaphoreType.DMA,
        pltpu.SemaphoreType.DMA,
        pltpu.SemaphoreType.DMA,
        pltpu.SemaphoreType.DMA,
    ]
    if with_counts:
        scratch += [
            pltpu.VMEM((CH,), jnp.float32),
            pltpu.VMEM((RLEN,), jnp.float32),
            pltpu.VMEM_SHARED((N,), jnp.float32),
        ]

    def body(table_hbm, ei_hbm, *refs):
        refs = list(refs)
        acc_out = refs.pop(0)
        if with_counts:
            cnt_out = refs.pop(0)
        (eib0, eib1, srcb0, srcb1, dstb0, dstb1, rows0, rows1, z2d, acc_sh,
         semg0, semg1, sems0, sems1, semi0, semi1, semz) = refs[:17]
        refs = refs[17:]
        if with_counts:
            ones_v, zrow, cnt_sh = refs
        c = lax.axis_index("c")
        s = lax.axis_index("s")
        wid = c * NS + s
        nf = Q + jnp.where(wid < R_, 1, 0)
        ch0 = wid * Q + jnp.minimum(wid, R_)

        def start_eib(i, eib, semi):
            pltpu.async_copy(ei_hbm.at[:, pl.ds((ch0 + i) * CH, CH)], eib,
                             semi)

        def wait_eib(eib, semi):
            pltpu.make_async_copy(ei_hbm.at[:, pl.ds(0, CH)], eib,
                                  semi).wait()

        def copy_idx(eib, srcb, dstb):
            for j in range(CH // LANES):
                sl = pl.ds(j * LANES, LANES)
                srcb[sl] = eib[0, sl]
                dstb[sl] = eib[1, sl]

        def start_gather(srcb, rows, semg):
            pltpu.async_copy(table_hbm.at[srcb], rows, semg)

        def wait_gather(rows, semg):
            pltpu.make_async_copy(table_hbm.at[pl.ds(0, CH)], rows,
                                  semg).wait()

        def start_scatter(rows, dstb, sems):
            pltpu.async_copy(rows, acc_sh.at[dstb], sems, add=True)
            if with_counts:
                pltpu.async_copy(ones_v, cnt_sh.at[dstb], sems, add=True)

        def wait_scatter(rows, dstb, sems):
            pltpu.make_async_copy(rows, acc_sh.at[dstb], sems).wait()
            if with_counts:
                pltpu.make_async_copy(ones_v, cnt_sh.at[dstb], sems).wait()

        if PQ > 0:
            pltpu.sync_copy(ei_hbm.at[:, pl.ds(ch0 * CH, CH)], eib0)
            copy_idx(eib0, srcb0, dstb0)
            start_gather(srcb0, rows0, semg0)
            start_eib(1, eib1, semi1)

        @pl.loop(0, ZR)
        def _(r):
            for j in range(D // LANES):
                z2d[r, pl.ds(j * LANES, LANES)] = jnp.zeros(
                    (LANES,), jnp.float32)

        row0 = s * RSTEP

        @pl.loop(0, NZ)
        def _(k):
            pltpu.async_copy(z2d, acc_sh.at[pl.ds(row0 + k * ZR, ZR)],
                             semz)

        @pl.loop(0, NZ)
        def _(k):
            pltpu.make_async_copy(z2d, acc_sh.at[pl.ds(row0 + k * ZR, ZR)],
                                  semz).wait()

        if with_counts:
            @pl.loop(0, RLEN // LANES)
            def _(i):
                zrow[pl.ds(i * LANES, LANES)] = jnp.zeros(
                    (LANES,), jnp.float32)

            @pl.loop(0, CH // LANES)
            def _(i):
                ones_v[pl.ds(i * LANES, LANES)] = jnp.ones(
                    (LANES,), jnp.float32)
            pltpu.sync_copy(zrow, cnt_sh.at[pl.ds(row0, RLEN)])

        plsc.subcore_barrier()

        if PQ > 0:
            @pl.loop(0, PQ)
            def _(k):
                a = k * 2
                wait_gather(rows0, semg0)

                @pl.when(k > 0)
                def _():
                    wait_scatter(rows1, dstb1, sems1)

                wait_eib(eib1, semi1)
                copy_idx(eib1, srcb1, dstb1)
                start_gather(srcb1, rows1, semg1)

                @pl.when(a + 2 < 2 * PQ)
                def _():
                    start_eib(a + 2, eib0, semi0)

                start_scatter(rows0, dstb0, sems0)

                wait_gather(rows1, semg1)
                wait_scatter(rows0, dstb0, sems0)

                @pl.when(a + 2 < 2 * PQ)
                def _():
                    wait_eib(eib0, semi0)
                    copy_idx(eib0, srcb0, dstb0)
                    start_gather(srcb0, rows0, semg0)

                @pl.when(a + 3 < 2 * PQ)
                def _():
                    start_eib(a + 3, eib1, semi1)

                start_scatter(rows1, dstb1, sems1)

            wait_scatter(rows1, dstb1, sems1)

        for t in range(EXTRA):
            @pl.when(2 * PQ + t < nf)
            def _():
                i = 2 * PQ + t
                pltpu.sync_copy(ei_hbm.at[:, pl.ds((ch0 + i) * CH, CH)],
                                eib0)
                copy_idx(eib0, srcb0, dstb0)
                start_gather(srcb0, rows0, semg0)
                wait_gather(rows0, semg0)
                pltpu.sync_copy(rows0, acc_sh.at[dstb0], add=True)
                if with_counts:
                    pltpu.sync_copy(ones_v, cnt_sh.at[dstb0], add=True)

        plsc.subcore_barrier()

        pub = [(rows0, semg0), (rows1, semg1)]
        for k in range(NPUB):
            buf, sem = pub[k % 2]
            lo = row0 + k * PB
            if k >= 2:
                pltpu.make_async_copy(
                    buf.at[pl.ds(0, PB)],
                    acc_out.at[c, pl.ds(row0 + (k - 2) * PB, PB)],
                    sem).wait()
            pltpu.sync_copy(acc_sh.at[pl.ds(lo, PB)], buf.at[pl.ds(0, PB)])
            pltpu.async_copy(buf.at[pl.ds(0, PB)],
                             acc_out.at[c, pl.ds(lo, PB)], sem)
        for k in range(max(0, NPUB - 2), NPUB):
            buf, sem = pub[k % 2]
            pltpu.make_async_copy(buf.at[pl.ds(0, PB)],
                                  acc_out.at[c, pl.ds(row0 + k * PB, PB)],
                                  sem).wait()

        if with_counts:
            pltpu.sync_copy(cnt_sh.at[pl.ds(row0, RLEN)], zrow)
            pltpu.sync_copy(zrow, cnt_out.at[pl.ds(c * N + row0, RLEN)])

    return pl.kernel(body, out_type=tuple(out_type), mesh=mesh,
                     scratch_types=scratch)


def _make_tc_layer1(N, D, H):
    R = _largest_div(N, 1024, 8)

    def body(acc, cnt, x, Wl, Wr, b1, gamma, beta, mu, var, out, inv_out):
        cntb = cnt[0] + cnt[1]
        inv = 1.0 / jnp.maximum(cntb, 1.0)
        inv_out[...] = inv
        mean = (acc[0] + acc[1]) * inv
        dn = (((1,), (1,)), ((), ()))
        t = lax.dot_general(mean, Wl[...], dn,
                            preferred_element_type=jnp.float32)
        t = t + lax.dot_general(x[...], Wr[...], dn,
                                preferred_element_type=jnp.float32)
        sc = gamma[...] * lax.rsqrt(var[...] + 1e-5)
        t = (t + b1[...] - mu[...]) * sc + beta[...]
        out[...] = jnp.maximum(t, 0.0)

    vec = pl.BlockSpec((1, H), lambda i: (0, 0))
    return pl.pallas_call(
        body,
        grid=(N // R,),
        in_specs=[
            pl.BlockSpec((2, R, D), lambda i: (0, i, 0)),
            pl.BlockSpec((2, R, 1), lambda i: (0, i, 0)),
            pl.BlockSpec((R, D), lambda i: (i, 0)),
            pl.BlockSpec((H, D), lambda i: (0, 0)),
            pl.BlockSpec((H, D), lambda i: (0, 0)),
            vec, vec, vec, vec, vec,
        ],
        out_specs=[
            pl.BlockSpec((R, H), lambda i: (i, 0)),
            pl.BlockSpec((R, 1), lambda i: (i, 0)),
        ],
        out_shape=[
            jax.ShapeDtypeStruct((N, H), jnp.float32),
            jax.ShapeDtypeStruct((N, 1), jnp.float32),
        ],
    )


def _make_tc_layer2(N, H, Z):
    R = _largest_div(N, 1024, 8)

    def body(acc, inv, h, Wl, Wr, bb, out):
        mean = (acc[0] + acc[1]) * inv[...]
        dn = (((1,), (1,)), ((), ()))
        t = lax.dot_general(mean, Wl[...], dn,
                            preferred_element_type=jnp.float32)
        t = t + lax.dot_general(h[...], Wr[...], dn,
                                preferred_element_type=jnp.float32)
        out[...] = t + bb[...]

    return pl.pallas_call(
        body,
        grid=(N // R,),
        in_specs=[
            pl.BlockSpec((2, R, H), lambda i: (0, i, 0)),
            pl.BlockSpec((R, 1), lambda i: (i, 0)),
            pl.BlockSpec((R, H), lambda i: (i, 0)),
            pl.BlockSpec((Z, H), lambda i: (0, 0)),
            pl.BlockSpec((Z, H), lambda i: (0, 0)),
            pl.BlockSpec((1, Z), lambda i: (0, 0)),
        ],
        out_specs=pl.BlockSpec((R, Z), lambda i: (i, 0)),
        out_shape=jax.ShapeDtypeStruct((N, Z), jnp.float32),
    )


def kernel(x, x_edge_index, W1l, b1, W1r, bn_gamma, bn_beta, bn_mean, bn_var,
           W2l, b2, W2r):
    N, D = x.shape
    E = x_edge_index.shape[1]
    H = W1l.shape[0]
    Z = W2l.shape[0]

    sc_agg1 = _make_sc_agg(N, E, D, with_counts=True)
    sc_agg2 = _make_sc_agg(N, E, H, with_counts=False)
    tc1 = _make_tc_layer1(N, D, H)
    tc2 = _make_tc_layer2(N, H, Z)

    acc1, cntp = sc_agg1(x, x_edge_index)
    cnt3 = cntp.reshape(NC, N, 1)

    h, inv = tc1(acc1, cnt3, x, W1l, W1r, b1.reshape(1, H),
                 bn_gamma.reshape(1, H), bn_beta.reshape(1, H),
                 bn_mean.reshape(1, H), bn_var.reshape(1, H))

    (acc2,) = sc_agg2(h, x_edge_index)
    z = tc2(acc2, inv, h, W2l, W2r, b2.reshape(1, Z))
    return z


# speedup vs baseline: 1.0253x; 1.0253x over previous
import jax
import jax.numpy as jnp
from jax import lax
from jax.experimental import pallas as pl
from jax.experimental.pallas import tpu as pltpu
from jax.experimental.pallas import tpu_sc as plsc

NC = 2
NS = 16
NW = NC * NS
LANES = 16
CH = 128


def _largest_div(n, limit, mult):
    best = mult
    for d in range(mult, limit + 1, mult):
        if n % d == 0:
            best = d
    return best


def _make_sc_agg(N, E, D, with_counts):
    NCHT = E // CH
    Q, R_ = divmod(NCHT, NW)
    PQ = Q // 2
    EXTRA = (Q - 2 * PQ) + (1 if R_ else 0)
    RSTEP = (N // NS) // 8 * 8
    RLEN = N - (NS - 1) * RSTEP
    ZR = _largest_div(RLEN, 16, 8)
    NZ = RLEN // ZR
    PB = _largest_div(RLEN, CH, 8)
    NPUB = RLEN // PB

    mesh = plsc.VectorSubcoreMesh(core_axis_name="c", subcore_axis_name="s",
                                  num_cores=NC, num_subcores=NS)

    out_type = [jax.ShapeDtypeStruct((NC, N, D), jnp.float32)]
    if with_counts:
        out_type.append(jax.ShapeDtypeStruct((NC * N,), jnp.float32))

    scratch = [
        pltpu.VMEM((2, CH), jnp.int32),
        pltpu.VMEM((2, CH), jnp.int32),
        pltpu.VMEM((CH,), jnp.int32),
        pltpu.VMEM((CH,), jnp.int32),
        pltpu.VMEM((CH,), jnp.int32),
        pltpu.VMEM((CH,), jnp.int32),
        pltpu.VMEM((CH, D), jnp.float32),
        pltpu.VMEM((CH, D), jnp.float32),
        pltpu.VMEM((ZR, D), jnp.float32),
        pltpu.VMEM_SHARED((N, D), jnp.float32),
        pltpu.SemaphoreType.DMA,
        pltpu.SemaphoreType.DMA,
        pltpu.SemaphoreType.DMA,
        pltpu.SemaphoreType.DMA,
        pltpu.SemaphoreType.DMA,
        pltpu.SemaphoreType.DMA,
        pltpu.SemaphoreType.DMA,
    ]
    if with_counts:
        scratch += [
            pltpu.VMEM((CH,), jnp.float32),
            pltpu.VMEM((RLEN,), jnp.float32),
            pltpu.VMEM_SHARED((N,), jnp.float32),
        ]

    def body(table_hbm, ei_hbm, *refs):
        refs = list(refs)
        acc_out = refs.pop(0)
        if with_counts:
            cnt_out = refs.pop(0)
        (eib0, eib1, srcb0, srcb1, dstb0, dstb1, rows0, rows1, z2d, acc_sh,
         semg0, semg1, sems0, sems1, semi0, semi1, semz) = refs[:17]
        refs = refs[17:]
        if with_counts:
            ones_v, zrow, cnt_sh = refs
        c = lax.axis_index("c")
        s = lax.axis_index("s")
        wid = c * NS + s
        nf = Q + jnp.where(wid < R_, 1, 0)
        ch0 = wid * Q + jnp.minimum(wid, R_)

        def start_eib(i, eib, semi):
            pltpu.async_copy(ei_hbm.at[:, pl.ds((ch0 + i) * CH, CH)], eib,
                             semi)

        def wait_eib(eib, semi):
            pltpu.make_async_copy(ei_hbm.at[:, pl.ds(0, CH)], eib,
                                  semi).wait()

        def copy_idx(eib, srcb, dstb):
            for j in range(CH // LANES):
                sl = pl.ds(j * LANES, LANES)
                srcb[sl] = eib[0, sl]
                dstb[sl] = eib[1, sl]

        def start_gather(srcb, rows, semg):
            pltpu.async_copy(table_hbm.at[srcb], rows, semg)

        def wait_gather(rows, semg):
            pltpu.make_async_copy(table_hbm.at[pl.ds(0, CH)], rows,
                                  semg).wait()

        def start_scatter(rows, dstb, sems):
            pltpu.async_copy(rows, acc_sh.at[dstb], sems, add=True)
            if with_counts:
                pltpu.async_copy(ones_v, cnt_sh.at[dstb], sems, add=True)

        def wait_scatter(rows, dstb, sems):
            pltpu.make_async_copy(rows, acc_sh.at[dstb], sems).wait()
            if with_counts:
                pltpu.make_async_copy(ones_v, cnt_sh.at[dstb], sems).wait()

        if PQ > 0:
            pltpu.sync_copy(ei_hbm.at[:, pl.ds(ch0 * CH, CH)], eib0)
            copy_idx(eib0, srcb0, dstb0)
            start_gather(srcb0, rows0, semg0)
            start_eib(1, eib1, semi1)

        @pl.loop(0, ZR)
        def _(r):
            for j in range(D // LANES):
                z2d[r, pl.ds(j * LANES, LANES)] = jnp.zeros(
                    (LANES,), jnp.float32)

        row0 = s * RSTEP

        @pl.loop(0, NZ)
        def _(k):
            pltpu.async_copy(z2d, acc_sh.at[pl.ds(row0 + k * ZR, ZR)],
                             semz)

        @pl.loop(0, NZ)
        def _(k):
            pltpu.make_async_copy(z2d, acc_sh.at[pl.ds(row0 + k * ZR, ZR)],
                                  semz).wait()

        if with_counts:
            @pl.loop(0, RLEN // LANES)
            def _(i):
                zrow[pl.ds(i * LANES, LANES)] = jnp.zeros(
                    (LANES,), jnp.float32)

            @pl.loop(0, CH // LANES)
            def _(i):
                ones_v[pl.ds(i * LANES, LANES)] = jnp.ones(
                    (LANES,), jnp.float32)
            pltpu.sync_copy(zrow, cnt_sh.at[pl.ds(row0, RLEN)])

        plsc.subcore_barrier()

        if PQ > 0:
            @pl.loop(0, PQ)
            def _(k):
                a = k * 2
                wait_gather(rows0, semg0)

                @pl.when(k > 0)
                def _():
                    wait_scatter(rows1, dstb1, sems1)

                wait_eib(eib1, semi1)
                copy_idx(eib1, srcb1, dstb1)
                start_gather(srcb1, rows1, semg1)

                @pl.when(a + 2 < 2 * PQ)
                def _():
                    start_eib(a + 2, eib0, semi0)

                start_scatter(rows0, dstb0, sems0)

                wait_gather(rows1, semg1)
                wait_scatter(rows0, dstb0, sems0)

                @pl.when(a + 2 < 2 * PQ)
                def _():
                    wait_eib(eib0, semi0)
                    copy_idx(eib0, srcb0, dstb0)
                    start_gather(srcb0, rows0, semg0)

                @pl.when(a + 3 < 2 * PQ)
                def _():
                    start_eib(a + 3, eib1, semi1)

                start_scatter(rows1, dstb1, sems1)

            wait_scatter(rows1, dstb1, sems1)

        for t in range(EXTRA):
            @pl.when(2 * PQ + t < nf)
            def _():
                i = 2 * PQ + t
                pltpu.sync_copy(ei_hbm.at[:, pl.ds((ch0 + i) * CH, CH)],
                                eib0)
                copy_idx(eib0, srcb0, dstb0)
                start_gather(srcb0, rows0, semg0)
                wait_gather(rows0, semg0)
                pltpu.sync_copy(rows0, acc_sh.at[dstb0], add=True)
                if with_counts:
                    pltpu.sync_copy(ones_v, cnt_sh.at[dstb0], add=True)

        plsc.subcore_barrier()

        pub = [(rows0, semg0), (rows1, semg1)]
        for k in range(NPUB):
            buf, sem = pub[k % 2]
            lo = row0 + k * PB
            if k >= 2:
                pltpu.make_async_copy(
                    buf.at[pl.ds(0, PB)],
                    acc_out.at[c, pl.ds(row0 + (k - 2) * PB, PB)],
                    sem).wait()
            pltpu.sync_copy(acc_sh.at[pl.ds(lo, PB)], buf.at[pl.ds(0, PB)])
            pltpu.async_copy(buf.at[pl.ds(0, PB)],
                             acc_out.at[c, pl.ds(lo, PB)], sem)
        for k in range(max(0, NPUB - 2), NPUB):
            buf, sem = pub[k % 2]
            pltpu.make_async_copy(buf.at[pl.ds(0, PB)],
                                  acc_out.at[c, pl.ds(row0 + k * PB, PB)],
                                  sem).wait()

        if with_counts:
            pltpu.sync_copy(cnt_sh.at[pl.ds(row0, RLEN)], zrow)
            pltpu.sync_copy(zrow, cnt_out.at[pl.ds(c * N + row0, RLEN)])

    return pl.kernel(body, out_type=tuple(out_type), mesh=mesh,
                     scratch_types=scratch)


def _make_tc_layer1(N, D, H):
    R = _largest_div(N, 1024, 8)

    def body(acc, invb, x, Wl, Wr, b1, gamma, beta, mu, var, out):
        mean = (acc[0] + acc[1]) * invb[...]
        dn = (((1,), (1,)), ((), ()))
        t = lax.dot_general(mean, Wl[...], dn,
                            preferred_element_type=jnp.float32)
        t = t + lax.dot_general(x[...], Wr[...], dn,
                                preferred_element_type=jnp.float32)
        sc = gamma[...] * lax.rsqrt(var[...] + 1e-5)
        t = (t + b1[...] - mu[...]) * sc + beta[...]
        out[...] = jnp.maximum(t, 0.0)

    vec = pl.BlockSpec((1, H), lambda i: (0, 0))
    return pl.pallas_call(
        body,
        grid=(N // R,),
        in_specs=[
            pl.BlockSpec((2, R, D), lambda i: (0, i, 0)),
            pl.BlockSpec((R, D), lambda i: (i, 0)),
            pl.BlockSpec((R, D), lambda i: (i, 0)),
            pl.BlockSpec((H, D), lambda i: (0, 0)),
            pl.BlockSpec((H, D), lambda i: (0, 0)),
            vec, vec, vec, vec, vec,
        ],
        out_specs=pl.BlockSpec((R, H), lambda i: (i, 0)),
        out_shape=jax.ShapeDtypeStruct((N, H), jnp.float32),
    )


def _make_tc_layer2(N, H, Z):
    R = _largest_div(N, 1024, 8)

    def body(acc, invb, h, Wl, Wr, bb, out):
        mean = (acc[0] + acc[1]) * invb[...]
        dn = (((1,), (1,)), ((), ()))
        t = lax.dot_general(mean, Wl[...], dn,
                            preferred_element_type=jnp.float32)
        t = t + lax.dot_general(h[...], Wr[...], dn,
                                preferred_element_type=jnp.float32)
        out[...] = t + bb[...]

    return pl.pallas_call(
        body,
        grid=(N // R,),
        in_specs=[
            pl.BlockSpec((2, R, H), lambda i: (0, i, 0)),
            pl.BlockSpec((R, H), lambda i: (i, 0)),
            pl.BlockSpec((R, H), lambda i: (i, 0)),
            pl.BlockSpec((Z, H), lambda i: (0, 0)),
            pl.BlockSpec((Z, H), lambda i: (0, 0)),
            pl.BlockSpec((1, Z), lambda i: (0, 0)),
        ],
        out_specs=pl.BlockSpec((R, Z), lambda i: (i, 0)),
        out_shape=jax.ShapeDtypeStruct((N, Z), jnp.float32),
    )


def kernel(x, x_edge_index, W1l, b1, W1r, bn_gamma, bn_beta, bn_mean, bn_var,
           W2l, b2, W2r):
    N, D = x.shape
    E = x_edge_index.shape[1]
    H = W1l.shape[0]
    Z = W2l.shape[0]

    sc_agg1 = _make_sc_agg(N, E, D, with_counts=True)
    sc_agg2 = _make_sc_agg(N, E, H, with_counts=False)
    tc1 = _make_tc_layer1(N, D, H)
    tc2 = _make_tc_layer2(N, H, Z)

    acc1, cntp = sc_agg1(x, x_edge_index)
    cnt2 = cntp.reshape(NC, N)
    invb = jnp.broadcast_to(
        (1.0 / jnp.maximum(cnt2[0] + cnt2[1], 1.0))[:, None], (N, D))

    h = tc1(acc1, invb, x, W1l, W1r, b1.reshape(1, H),
            bn_gamma.reshape(1, H), bn_beta.reshape(1, H),
            bn_mean.reshape(1, H), bn_var.reshape(1, H))

    (acc2,) = sc_agg2(h, x_edge_index)
    z = tc2(acc2, invb, h, W2l, W2r, b2.reshape(1, Z))
    return z


# speedup vs baseline: 1.0292x; 1.0038x over previous
import jax
import jax.numpy as jnp
from jax import lax
from jax.experimental import pallas as pl
from jax.experimental.pallas import tpu as pltpu
from jax.experimental.pallas import tpu_sc as plsc

NC = 2
NS = 16
NW = NC * NS
LANES = 16
CH = 128


def _largest_div(n, limit, mult):
    best = mult
    for d in range(mult, limit + 1, mult):
        if n % d == 0:
            best = d
    return best


def _make_sc_agg(N, E, D, with_counts):
    NCHT = E // CH
    Q, R_ = divmod(NCHT, NW)
    PQ = Q // 2
    EXTRA = (Q - 2 * PQ) + (1 if R_ else 0)
    RSTEP = (N // NS) // 8 * 8
    RLEN = N - (NS - 1) * RSTEP
    ZR = _largest_div(RLEN, 16, 8)
    NZ = RLEN // ZR
    PB = _largest_div(RLEN, CH, 8)
    NPUB = RLEN // PB

    mesh = plsc.VectorSubcoreMesh(core_axis_name="c", subcore_axis_name="s",
                                  num_cores=NC, num_subcores=NS)

    out_type = [jax.ShapeDtypeStruct((NC, N, D), jnp.float32)]
    if with_counts:
        out_type.append(jax.ShapeDtypeStruct((NC * N,), jnp.float32))

    scratch = [
        pltpu.VMEM((2, CH), jnp.int32),
        pltpu.VMEM((2, CH), jnp.int32),
        pltpu.VMEM((CH,), jnp.int32),
        pltpu.VMEM((CH,), jnp.int32),
        pltpu.VMEM((CH,), jnp.int32),
        pltpu.VMEM((CH,), jnp.int32),
        pltpu.VMEM((CH, D), jnp.float32),
        pltpu.VMEM((CH, D), jnp.float32),
        pltpu.VMEM((ZR, D), jnp.float32),
        pltpu.VMEM_SHARED((N, D), jnp.float32),
        pltpu.SemaphoreType.DMA,
        pltpu.SemaphoreType.DMA,
        pltpu.SemaphoreType.DMA,
        pltpu.SemaphoreType.DMA,
        pltpu.SemaphoreType.DMA,
        pltpu.SemaphoreType.DMA,
        pltpu.SemaphoreType.DMA,
    ]
    if with_counts:
        scratch += [
            pltpu.VMEM((CH,), jnp.float32),
            pltpu.VMEM((RLEN,), jnp.float32),
            pltpu.VMEM_SHARED((N,), jnp.float32),
        ]

    def body(table_hbm, ei_hbm, *refs):
        refs = list(refs)
        acc_out = refs.pop(0)
        if with_counts:
            cnt_out = refs.pop(0)
        (eib0, eib1, srcb0, srcb1, dstb0, dstb1, rows0, rows1, z2d, acc_sh,
         semg0, semg1, sems0, sems1, semi0, semi1, semz) = refs[:17]
        refs = refs[17:]
        if with_counts:
            ones_v, zrow, cnt_sh = refs
        c = lax.axis_index("c")
        s = lax.axis_index("s")
        wid = c * NS + s
        nf = Q + jnp.where(wid < R_, 1, 0)
        ch0 = wid * Q + jnp.minimum(wid, R_)

        def start_eib(i, eib, semi):
            pltpu.async_copy(ei_hbm.at[:, pl.ds((ch0 + i) * CH, CH)], eib,
                             semi)

        def wait_eib(eib, semi):
            pltpu.make_async_copy(ei_hbm.at[:, pl.ds(0, CH)], eib,
                                  semi).wait()

        def copy_idx(eib, srcb, dstb):
            for j in range(CH // LANES):
                sl = pl.ds(j * LANES, LANES)
                dstb[sl] = eib[1, sl]

        def start_gather(eib, rows, semg):
            pltpu.async_copy(table_hbm.at[eib.at[0]], rows, semg)

        def wait_gather(rows, semg):
            pltpu.make_async_copy(table_hbm.at[pl.ds(0, CH)], rows,
                                  semg).wait()

        def start_scatter(rows, dstb, sems):
            pltpu.async_copy(rows, acc_sh.at[dstb], sems, add=True)
            if with_counts:
                pltpu.async_copy(ones_v, cnt_sh.at[dstb], sems, add=True)

        def wait_scatter(rows, dstb, sems):
            pltpu.make_async_copy(rows, acc_sh.at[dstb], sems).wait()
            if with_counts:
                pltpu.make_async_copy(ones_v, cnt_sh.at[dstb], sems).wait()

        if PQ > 0:
            pltpu.sync_copy(ei_hbm.at[:, pl.ds(ch0 * CH, CH)], eib0)
            copy_idx(eib0, srcb0, dstb0)
            start_gather(eib0, rows0, semg0)
            start_eib(1, eib1, semi1)

        @pl.loop(0, ZR)
        def _(r):
            for j in range(D // LANES):
                z2d[r, pl.ds(j * LANES, LANES)] = jnp.zeros(
                    (LANES,), jnp.float32)

        row0 = s * RSTEP

        @pl.loop(0, NZ)
        def _(k):
            pltpu.async_copy(z2d, acc_sh.at[pl.ds(row0 + k * ZR, ZR)],
                             semz)

        @pl.loop(0, NZ)
        def _(k):
            pltpu.make_async_copy(z2d, acc_sh.at[pl.ds(row0 + k * ZR, ZR)],
                                  semz).wait()

        if with_counts:
            @pl.loop(0, RLEN // LANES)
            def _(i):
                zrow[pl.ds(i * LANES, LANES)] = jnp.zeros(
                    (LANES,), jnp.float32)

            @pl.loop(0, CH // LANES)
            def _(i):
                ones_v[pl.ds(i * LANES, LANES)] = jnp.ones(
                    (LANES,), jnp.float32)
            pltpu.sync_copy(zrow, cnt_sh.at[pl.ds(row0, RLEN)])

        plsc.subcore_barrier()

        if PQ > 0:
            @pl.loop(0, PQ)
            def _(k):
                a = k * 2
                wait_gather(rows0, semg0)

                @pl.when(k > 0)
                def _():
                    wait_scatter(rows1, dstb1, sems1)

                wait_eib(eib1, semi1)
                copy_idx(eib1, srcb1, dstb1)
                start_gather(eib1, rows1, semg1)

                @pl.when(a + 2 < 2 * PQ)
                def _():
                    start_eib(a + 2, eib0, semi0)

                start_scatter(rows0, dstb0, sems0)

                wait_gather(rows1, semg1)
                wait_scatter(rows0, dstb0, sems0)

                @pl.when(a + 2 < 2 * PQ)
                def _():
                    wait_eib(eib0, semi0)
                    copy_idx(eib0, srcb0, dstb0)
                    start_gather(eib0, rows0, semg0)

                @pl.when(a + 3 < 2 * PQ)
                def _():
                    start_eib(a + 3, eib1, semi1)

                start_scatter(rows1, dstb1, sems1)

            wait_scatter(rows1, dstb1, sems1)

        for t in range(EXTRA):
            @pl.when(2 * PQ + t < nf)
            def _():
                i = 2 * PQ + t
                pltpu.sync_copy(ei_hbm.at[:, pl.ds((ch0 + i) * CH, CH)],
                                eib0)
                copy_idx(eib0, srcb0, dstb0)
                start_gather(eib0, rows0, semg0)
                wait_gather(rows0, semg0)
                pltpu.sync_copy(rows0, acc_sh.at[dstb0], add=True)
                if with_counts:
                    pltpu.sync_copy(ones_v, cnt_sh.at[dstb0], add=True)

        plsc.subcore_barrier()

        pub = [(rows0, semg0), (rows1, semg1)]
        for k in range(NPUB):
            buf, sem = pub[k % 2]
            lo = row0 + k * PB
            if k >= 2:
                pltpu.make_async_copy(
                    buf.at[pl.ds(0, PB)],
                    acc_out.at[c, pl.ds(row0 + (k - 2) * PB, PB)],
                    sem).wait()
            pltpu.sync_copy(acc_sh.at[pl.ds(lo, PB)], buf.at[pl.ds(0, PB)])
            pltpu.async_copy(buf.at[pl.ds(0, PB)],
                             acc_out.at[c, pl.ds(lo, PB)], sem)
        for k in range(max(0, NPUB - 2), NPUB):
            buf, sem = pub[k % 2]
            pltpu.make_async_copy(buf.at[pl.ds(0, PB)],
                                  acc_out.at[c, pl.ds(row0 + k * PB, PB)],
                                  sem).wait()

        if with_counts:
            pltpu.sync_copy(cnt_sh.at[pl.ds(row0, RLEN)], zrow)
            pltpu.sync_copy(zrow, cnt_out.at[pl.ds(c * N + row0, RLEN)])

    return pl.kernel(body, out_type=tuple(out_type), mesh=mesh,
                     scratch_types=scratch)


def _make_tc_layer1(N, D, H):
    R = _largest_div(N, 1024, 8)

    def body(acc, invb, x, Wl, Wr, b1, gamma, beta, mu, var, out):
        mean = (acc[0] + acc[1]) * invb[...]
        dn = (((1,), (1,)), ((), ()))
        t = lax.dot_general(mean, Wl[...], dn,
                            preferred_element_type=jnp.float32)
        t = t + lax.dot_general(x[...], Wr[...], dn,
                                preferred_element_type=jnp.float32)
        sc = gamma[...] * lax.rsqrt(var[...] + 1e-5)
        t = (t + b1[...] - mu[...]) * sc + beta[...]
        out[...] = jnp.maximum(t, 0.0)

    vec = pl.BlockSpec((1, H), lambda i: (0, 0))
    return pl.pallas_call(
        body,
        grid=(N // R,),
        in_specs=[
            pl.BlockSpec((2, R, D), lambda i: (0, i, 0)),
            pl.BlockSpec((R, D), lambda i: (i, 0)),
            pl.BlockSpec((R, D), lambda i: (i, 0)),
            pl.BlockSpec((H, D), lambda i: (0, 0)),
            pl.BlockSpec((H, D), lambda i: (0, 0)),
            vec, vec, vec, vec, vec,
        ],
        out_specs=pl.BlockSpec((R, H), lambda i: (i, 0)),
        out_shape=jax.ShapeDtypeStruct((N, H), jnp.float32),
    )


def _make_tc_layer2(N, H, Z):
    R = _largest_div(N, 1024, 8)

    def body(acc, invb, h, Wl, Wr, bb, out):
        mean = (acc[0] + acc[1]) * invb[...]
        dn = (((1,), (1,)), ((), ()))
        t = lax.dot_general(mean, Wl[...], dn,
                            preferred_element_type=jnp.float32)
        t = t + lax.dot_general(h[...], Wr[...], dn,
                                preferred_element_type=jnp.float32)
        out[...] = t + bb[...]

    return pl.pallas_call(
        body,
        grid=(N // R,),
        in_specs=[
            pl.BlockSpec((2, R, H), lambda i: (0, i, 0)),
            pl.BlockSpec((R, H), lambda i: (i, 0)),
            pl.BlockSpec((R, H), lambda i: (i, 0)),
            pl.BlockSpec((Z, H), lambda i: (0, 0)),
            pl.BlockSpec((Z, H), lambda i: (0, 0)),
            pl.BlockSpec((1, Z), lambda i: (0, 0)),
        ],
        out_specs=pl.BlockSpec((R, Z), lambda i: (i, 0)),
        out_shape=jax.ShapeDtypeStruct((N, Z), jnp.float32),
    )


def kernel(x, x_edge_index, W1l, b1, W1r, bn_gamma, bn_beta, bn_mean, bn_var,
           W2l, b2, W2r):
    N, D = x.shape
    E = x_edge_index.shape[1]
    H = W1l.shape[0]
    Z = W2l.shape[0]

    sc_agg1 = _make_sc_agg(N, E, D, with_counts=True)
    sc_agg2 = _make_sc_agg(N, E, H, with_counts=False)
    tc1 = _make_tc_layer1(N, D, H)
    tc2 = _make_tc_layer2(N, H, Z)

    acc1, cntp = sc_agg1(x, x_edge_index)
    cnt2 = cntp.reshape(NC, N)
    invb = jnp.broadcast_to(
        (1.0 / jnp.maximum(cnt2[0] + cnt2[1], 1.0))[:, None], (N, D))

    h = tc1(acc1, invb, x, W1l, W1r, b1.reshape(1, H),
            bn_gamma.reshape(1, H), bn_beta.reshape(1, H),
            bn_mean.reshape(1, H), bn_var.reshape(1, H))

    (acc2,) = sc_agg2(h, x_edge_index)
    z = tc2(acc2, invb, h, W2l, W2r, b2.reshape(1, Z))
    return z


# speedup vs baseline: 1.0338x; 1.0045x over previous
import jax
import jax.numpy as jnp
from jax import lax
from jax.experimental import pallas as pl
from jax.experimental.pallas import tpu as pltpu
from jax.experimental.pallas import tpu_sc as plsc

NC = 2
NS = 16
NW = NC * NS
LANES = 16
CH = 128


def _largest_div(n, limit, mult):
    best = mult
    for d in range(mult, limit + 1, mult):
        if n % d == 0:
            best = d
    return best


def _make_sc_agg(N, E, D, with_counts):
    NCHT = E // CH
    Q, R_ = divmod(NCHT, NW)
    PQ = Q // 2
    EXTRA = (Q - 2 * PQ) + (1 if R_ else 0)
    RSTEP = (N // NS) // 8 * 8
    RLEN = N - (NS - 1) * RSTEP
    ZR = _largest_div(RLEN, 16, 8)
    NZ = RLEN // ZR
    PB = _largest_div(RLEN, CH, 8)
    NPUB = RLEN // PB

    mesh = plsc.VectorSubcoreMesh(core_axis_name="c", subcore_axis_name="s",
                                  num_cores=NC, num_subcores=NS)

    out_type = [jax.ShapeDtypeStruct((NC, N, D), jnp.float32)]
    if with_counts:
        out_type.append(jax.ShapeDtypeStruct((NC * N,), jnp.float32))

    scratch = [
        pltpu.VMEM((2, CH), jnp.int32),
        pltpu.VMEM((2, CH), jnp.int32),
        pltpu.VMEM((CH,), jnp.int32),
        pltpu.VMEM((CH,), jnp.int32),
        pltpu.VMEM((CH, D), jnp.float32),
        pltpu.VMEM((CH, D), jnp.float32),
        pltpu.VMEM((ZR, D), jnp.float32),
        pltpu.VMEM_SHARED((N, D), jnp.float32),
        pltpu.SemaphoreType.DMA,
        pltpu.SemaphoreType.DMA,
        pltpu.SemaphoreType.DMA,
        pltpu.SemaphoreType.DMA,
        pltpu.SemaphoreType.DMA,
        pltpu.SemaphoreType.DMA,
        pltpu.SemaphoreType.DMA,
    ]
    if with_counts:
        scratch += [
            pltpu.VMEM((CH,), jnp.float32),
            pltpu.VMEM((RLEN,), jnp.float32),
            pltpu.VMEM_SHARED((N,), jnp.float32),
        ]

    def body(table_hbm, ei_hbm, *refs):
        refs = list(refs)
        acc_out = refs.pop(0)
        if with_counts:
            cnt_out = refs.pop(0)
        (eib0, eib1, dstb0, dstb1, rows0, rows1, z2d, acc_sh,
         semg0, semg1, sems0, sems1, semi0, semi1, semz) = refs[:15]
        refs = refs[15:]
        if with_counts:
            ones_v, zrow, cnt_sh = refs
        c = lax.axis_index("c")
        s = lax.axis_index("s")
        wid = c * NS + s
        nf = Q + jnp.where(wid < R_, 1, 0)
        ch0 = wid * Q + jnp.minimum(wid, R_)

        def start_eib(i, eib, semi):
            pltpu.async_copy(ei_hbm.at[:, pl.ds((ch0 + i) * CH, CH)], eib,
                             semi)

        def wait_eib(eib, semi):
            pltpu.make_async_copy(ei_hbm.at[:, pl.ds(0, CH)], eib,
                                  semi).wait()

        def copy_idx(eib, dstb):
            for j in range(CH // LANES):
                sl = pl.ds(j * LANES, LANES)
                dstb[sl] = eib[1, sl]

        def start_gather(eib, rows, semg):
            pltpu.async_copy(table_hbm.at[eib.at[0]], rows, semg)

        def wait_gather(rows, semg):
            pltpu.make_async_copy(table_hbm.at[pl.ds(0, CH)], rows,
                                  semg).wait()

        def start_scatter(rows, dstb, sems):
            pltpu.async_copy(rows, acc_sh.at[dstb], sems, add=True)
            if with_counts:
                pltpu.async_copy(ones_v, cnt_sh.at[dstb], sems, add=True)

        def wait_scatter(rows, dstb, sems):
            pltpu.make_async_copy(rows, acc_sh.at[dstb], sems).wait()
            if with_counts:
                pltpu.make_async_copy(ones_v, cnt_sh.at[dstb], sems).wait()

        if PQ > 0:
            pltpu.sync_copy(ei_hbm.at[:, pl.ds(ch0 * CH, CH)], eib0)
            copy_idx(eib0, dstb0)
            start_gather(eib0, rows0, semg0)
            start_eib(1, eib1, semi1)

        @pl.loop(0, ZR)
        def _(r):
            for j in range(D // LANES):
                z2d[r, pl.ds(j * LANES, LANES)] = jnp.zeros(
                    (LANES,), jnp.float32)

        row0 = s * RSTEP

        @pl.loop(0, NZ)
        def _(k):
            pltpu.async_copy(z2d, acc_sh.at[pl.ds(row0 + k * ZR, ZR)],
                             semz)

        @pl.loop(0, NZ)
        def _(k):
            pltpu.make_async_copy(z2d, acc_sh.at[pl.ds(row0 + k * ZR, ZR)],
                                  semz).wait()

        if with_counts:
            @pl.loop(0, RLEN // LANES)
            def _(i):
                zrow[pl.ds(i * LANES, LANES)] = jnp.zeros(
                    (LANES,), jnp.float32)

            @pl.loop(0, CH // LANES)
            def _(i):
                ones_v[pl.ds(i * LANES, LANES)] = jnp.ones(
                    (LANES,), jnp.float32)
            pltpu.sync_copy(zrow, cnt_sh.at[pl.ds(row0, RLEN)])

        plsc.subcore_barrier()

        if PQ > 0:
            @pl.loop(0, PQ)
            def _(k):
                a = k * 2
                wait_gather(rows0, semg0)

                @pl.when(k > 0)
                def _():
                    wait_scatter(rows1, dstb1, sems1)

                wait_eib(eib1, semi1)
                copy_idx(eib1, dstb1)
                start_gather(eib1, rows1, semg1)

                @pl.when(a + 2 < 2 * PQ)
                def _():
                    start_eib(a + 2, eib0, semi0)

                start_scatter(rows0, dstb0, sems0)

                wait_gather(rows1, semg1)
                wait_scatter(rows0, dstb0, sems0)

                @pl.when(a + 2 < 2 * PQ)
                def _():
                    wait_eib(eib0, semi0)
                    copy_idx(eib0, dstb0)
                    start_gather(eib0, rows0, semg0)

                @pl.when(a + 3 < 2 * PQ)
                def _():
                    start_eib(a + 3, eib1, semi1)

                start_scatter(rows1, dstb1, sems1)

            wait_scatter(rows1, dstb1, sems1)

        for t in range(EXTRA):
            @pl.when(2 * PQ + t < nf)
            def _():
                i = 2 * PQ + t
                pltpu.sync_copy(ei_hbm.at[:, pl.ds((ch0 + i) * CH, CH)],
                                eib0)
                copy_idx(eib0, dstb0)
                start_gather(eib0, rows0, semg0)
                wait_gather(rows0, semg0)
                pltpu.sync_copy(rows0, acc_sh.at[dstb0], add=True)
                if with_counts:
                    pltpu.sync_copy(ones_v, cnt_sh.at[dstb0], add=True)

        plsc.subcore_barrier()

        pub = [(rows0, semg0), (rows1, semg1)]
        for k in range(NPUB):
            buf, sem = pub[k % 2]
            lo = row0 + k * PB
            if k >= 2:
                pltpu.make_async_copy(
                    buf.at[pl.ds(0, PB)],
                    acc_out.at[c, pl.ds(row0 + (k - 2) * PB, PB)],
                    sem).wait()
            pltpu.sync_copy(acc_sh.at[pl.ds(lo, PB)], buf.at[pl.ds(0, PB)])
            pltpu.async_copy(buf.at[pl.ds(0, PB)],
                             acc_out.at[c, pl.ds(lo, PB)], sem)
        for k in range(max(0, NPUB - 2), NPUB):
            buf, sem = pub[k % 2]
            pltpu.make_async_copy(buf.at[pl.ds(0, PB)],
                                  acc_out.at[c, pl.ds(row0 + k * PB, PB)],
                                  sem).wait()

        if with_counts:
            pltpu.sync_copy(cnt_sh.at[pl.ds(row0, RLEN)], zrow)
            pltpu.sync_copy(zrow, cnt_out.at[pl.ds(c * N + row0, RLEN)])

    return pl.kernel(body, out_type=tuple(out_type), mesh=mesh,
                     scratch_types=scratch)


def _make_tc_layer1(N, D, H):
    R = _largest_div(N, 1024, 8)

    def body(acc, invb, x, Wl, Wr, b1, gamma, beta, mu, var, out):
        mean = (acc[0] + acc[1]) * invb[...]
        dn = (((1,), (1,)), ((), ()))
        t = lax.dot_general(mean, Wl[...], dn,
                            preferred_element_type=jnp.float32)
        t = t + lax.dot_general(x[...], Wr[...], dn,
                                preferred_element_type=jnp.float32)
        sc = gamma[...] * lax.rsqrt(var[...] + 1e-5)
        t = (t + b1[...] - mu[...]) * sc + beta[...]
        out[...] = jnp.maximum(t, 0.0)

    vec = pl.BlockSpec((1, H), lambda i: (0, 0))
    return pl.pallas_call(
        body,
        grid=(N // R,),
        in_specs=[
            pl.BlockSpec((2, R, D), lambda i: (0, i, 0)),
            pl.BlockSpec((R, D), lambda i: (i, 0)),
            pl.BlockSpec((R, D), lambda i: (i, 0)),
            pl.BlockSpec((H, D), lambda i: (0, 0)),
            pl.BlockSpec((H, D), lambda i: (0, 0)),
            vec, vec, vec, vec, vec,
        ],
        out_specs=pl.BlockSpec((R, H), lambda i: (i, 0)),
        out_shape=jax.ShapeDtypeStruct((N, H), jnp.float32),
    )


def _make_tc_layer2(N, H, Z):
    R = _largest_div(N, 1024, 8)

    def body(acc, invb, h, Wl, Wr, bb, out):
        mean = (acc[0] + acc[1]) * invb[...]
        dn = (((1,), (1,)), ((), ()))
        t = lax.dot_general(mean, Wl[...], dn,
                            preferred_element_type=jnp.float32)
        t = t + lax.dot_general(h[...], Wr[...], dn,
                                preferred_element_type=jnp.float32)
        out[...] = t + bb[...]

    return pl.pallas_call(
        body,
        grid=(N // R,),
        in_specs=[
            pl.BlockSpec((2, R, H), lambda i: (0, i, 0)),
            pl.BlockSpec((R, H), lambda i: (i, 0)),
            pl.BlockSpec((R, H), lambda i: (i, 0)),
            pl.BlockSpec((Z, H), lambda i: (0, 0)),
            pl.BlockSpec((Z, H), lambda i: (0, 0)),
            pl.BlockSpec((1, Z), lambda i: (0, 0)),
        ],
        out_specs=pl.BlockSpec((R, Z), lambda i: (i, 0)),
        out_shape=jax.ShapeDtypeStruct((N, Z), jnp.float32),
    )


def kernel(x, x_edge_index, W1l, b1, W1r, bn_gamma, bn_beta, bn_mean, bn_var,
           W2l, b2, W2r):
    N, D = x.shape
    E = x_edge_index.shape[1]
    H = W1l.shape[0]
    Z = W2l.shape[0]

    sc_agg1 = _make_sc_agg(N, E, D, with_counts=True)
    sc_agg2 = _make_sc_agg(N, E, H, with_counts=False)
    tc1 = _make_tc_layer1(N, D, H)
    tc2 = _make_tc_layer2(N, H, Z)

    acc1, cntp = sc_agg1(x, x_edge_index)
    cnt2 = cntp.reshape(NC, N)
    invb = jnp.broadcast_to(
        (1.0 / jnp.maximum(cnt2[0] + cnt2[1], 1.0))[:, None], (N, D))

    h = tc1(acc1, invb, x, W1l, W1r, b1.reshape(1, H),
            bn_gamma.reshape(1, H), bn_beta.reshape(1, H),
            bn_mean.reshape(1, H), bn_var.reshape(1, H))

    (acc2,) = sc_agg2(h, x_edge_index)
    z = tc2(acc2, invb, h, W2l, W2r, b2.reshape(1, Z))
    return z
